# baseline (speedup 1.0000x reference)
import functools

import jax
import jax.numpy as jnp
from jax import lax
from jax.experimental import pallas as pl
from jax.experimental.pallas import tpu as pltpu
from jax.experimental.pallas import tpu_sc as plsc

N = 8192
D = 128
DIM = 2048
NC = 2
NS = 16
NW = NC * NS
ROWS_PER_TILE = N // NW
NODES_PER_TILE = DIM // NW
PAD_ROWS = 8
ZERO_ROW = N
NSTAGE = 13
LANES = 16

_PREC = jax.lax.Precision.DEFAULT


def _dot_t(a, m):
    return jax.lax.dot_general(a, m, (((1,), (1,)), ((), ())),
                               preferred_element_type=jnp.float32,
                               precision=_PREC)


SHIFT_PAD = N // 2


def _tc_scan_body(msgs_ref, ibuf, w_ref, b_ref, xpad_ref,
                  xbuf, m_scr):
    w = w_ref[...]
    m_scr[...] = w
    xbuf[0, pl.ds(SHIFT_PAD, N), :] = _dot_t(msgs_ref[...], w) + b_ref[...]

    def _cond(carry):
        k, p, cont = carry
        return (k < NSTAGE) & (cont == 1)

    def _stage(carry):
        k, p, cont = carry
        q = 1 - p
        s = jax.lax.shift_left(1, k)
        m = m_scr[...]
        d = ibuf[pl.ds(SHIFT_PAD, N), :] - ibuf[pl.ds(SHIFT_PAD - s, N), :]
        mask = jnp.abs(d) < 0.5
        x = xbuf[p, pl.ds(SHIFT_PAD, N), :]
        xs = xbuf[p, pl.ds(SHIFT_PAD - s, N), :]
        xm = jnp.where(mask, xs, jnp.zeros_like(xs))
        xbuf[q, pl.ds(SHIFT_PAD, N), :] = x + _dot_t(xm, m)
        m_scr[...] = jax.lax.dot_general(m, m, (((1,), (0,)), ((), ())),
                                         preferred_element_type=jnp.float32,
                                         precision=_PREC)
        return k + 1, q, jnp.max(jnp.where(mask, 1, 0))

    _, q, _ = jax.lax.while_loop(_cond, _stage, (0, 0, 1))
    xpad_ref[pl.ds(0, N), :] = xbuf[q, pl.ds(SHIFT_PAD, N), :]
    xpad_ref[pl.ds(N, PAD_ROWS), :] = jnp.zeros((PAD_ROWS, D), jnp.float32)


def _make_tc_scan(interpret=False):
  return pl.pallas_call(
    _tc_scan_body,
    interpret=interpret,
    out_shape=jax.ShapeDtypeStruct((N + PAD_ROWS, D), jnp.float32),
    scratch_shapes=[
        pltpu.VMEM((2, SHIFT_PAD + N, D), jnp.float32),
        pltpu.VMEM((D, D), jnp.float32),
    ],
  )


_tc_scan = _make_tc_scan()


@functools.lru_cache(maxsize=None)
def _build_sc_gather():
    mesh = plsc.VectorSubcoreMesh(core_axis_name="c", subcore_axis_name="s")
    return functools.partial(
        pl.kernel,
        out_type=[
            jax.ShapeDtypeStruct((N, D), jnp.float32),
            jax.ShapeDtypeStruct((N,), jnp.int32),
            jax.ShapeDtypeStruct((SHIFT_PAD + N, D), jnp.float32),
        ],
        mesh=mesh,
        scratch_types=[
            pltpu.VMEM((ROWS_PER_TILE,), jnp.int32),
            pltpu.VMEM((ROWS_PER_TILE, D), jnp.float32),
            pltpu.VMEM((N,), jnp.int32),
            pltpu.VMEM((ROWS_PER_TILE,), jnp.int32),
            pltpu.VMEM((ROWS_PER_TILE, D), jnp.float32),
            pltpu.VMEM((8, D), jnp.float32),
            pltpu.SemaphoreType.DMA,
        ],
        compiler_params=pltpu.CompilerParams(needs_layout_passes=False),
    )(_sc_gather_body)


def _sc_gather_body(msg_hbm, order_hbm, index_hbm, msgs_hbm, idxs_hbm,
                    idxb_hbm, ord_v, rows_v, index_v, idxs_v, fbuf_v,
                    pad_v, sem):
    wid = lax.axis_index("s") * NC + lax.axis_index("c")
    base = wid * ROWS_PER_TILE
    pltpu.sync_copy(order_hbm.at[pl.ds(base, ROWS_PER_TILE)], ord_v)
    copies = []
    for j in range(ROWS_PER_TILE // 128):
        copies.append(pltpu.async_copy(
            msg_hbm.at[ord_v.at[pl.ds(j * 128, 128)]],
            rows_v.at[pl.ds(j * 128, 128)], sem))
    pltpu.sync_copy(index_hbm, index_v)
    for g in range(ROWS_PER_TILE // LANES):
        o16 = ord_v[pl.ds(g * LANES, LANES)]
        idxs_v[pl.ds(g * LANES, LANES)] = plsc.load_gather(index_v, [o16])
    neg = jnp.full((LANES,), -1.0, jnp.float32)
    for r in range(8):
        for u in range(D // LANES):
            pad_v[r, pl.ds(u * LANES, LANES)] = neg
    pad_rows = SHIFT_PAD // NW
    for c in range(pad_rows // 8):
        copies.append(pltpu.async_copy(
            pad_v, idxb_hbm.at[pl.ds(wid * pad_rows + c * 8, 8)], sem))
    for g in range(ROWS_PER_TILE // LANES):
        v16 = idxs_v[pl.ds(g * LANES, LANES)].astype(jnp.float32)
        for j in range(LANES):
            vec = jnp.full((LANES,), v16[j], jnp.float32)
            for u in range(D // LANES):
                fbuf_v[g * LANES + j, pl.ds(u * LANES, LANES)] = vec
    for c in copies:
        c.wait()
    pltpu.sync_copy(rows_v, msgs_hbm.at[pl.ds(base, ROWS_PER_TILE)])
    pltpu.sync_copy(idxs_v, idxs_hbm.at[pl.ds(base, ROWS_PER_TILE)])
    pltpu.sync_copy(fbuf_v,
                    idxb_hbm.at[pl.ds(SHIFT_PAD + base, ROWS_PER_TILE)])


@functools.lru_cache(maxsize=None)
def _build_sc_final():
    mesh = plsc.VectorSubcoreMesh(core_axis_name="c", subcore_axis_name="s")
    return functools.partial(
        pl.kernel,
        out_type=jax.ShapeDtypeStruct((DIM, D), jnp.float32),
        mesh=mesh,
        scratch_types=[
            pltpu.VMEM((N,), jnp.int32),
            pltpu.VMEM((NODES_PER_TILE,), jnp.int32),
            pltpu.VMEM((NODES_PER_TILE, D), jnp.float32),
            pltpu.SemaphoreType.DMA,
        ],
        compiler_params=pltpu.CompilerParams(needs_layout_passes=False),
    )(_sc_final_body)


def _sc_final_body(idxs_hbm, xpad_hbm, out_hbm, idx_v, fin_v, rows_v, sem):
    wid = lax.axis_index("s") * NC + lax.axis_index("c")
    base = wid * NODES_PER_TILE
    pltpu.sync_copy(idxs_hbm, idx_v)
    lane = lax.iota(jnp.int32, LANES)
    for g in range(NODES_PER_TILE // LANES):
        nid = base + g * LANES + lane
        lo = jnp.zeros((LANES,), jnp.int32)
        hi = jnp.full((LANES,), N, jnp.int32)
        for _ in range(NSTAGE):
            active = lo < hi
            mid = (lo + hi) >> 1
            midc = jnp.minimum(mid, jnp.full((LANES,), N - 1, jnp.int32))
            v = plsc.load_gather(idx_v, [midc])
            le = (v <= nid) & active
            gt = (v > nid) & active
            lo = jnp.where(le, mid + 1, lo)
            hi = jnp.where(gt, mid, hi)
        lp = lo - 1
        val = plsc.load_gather(idx_v, [jnp.maximum(lp, 0)])
        has = (lp >= 0) & (val == nid)
        fin_v[pl.ds(g * LANES, LANES)] = jnp.where(
            has, lp, jnp.full((LANES,), ZERO_ROW, jnp.int32))
    pltpu.async_copy(xpad_hbm.at[fin_v], rows_v, sem).wait()
    pltpu.sync_copy(rows_v, out_hbm.at[pl.ds(base, NODES_PER_TILE)])


def kernel(msg, index, t, dim_size, W, b):
    del dim_size
    order = jnp.lexsort((t, index)).astype(jnp.int32)
    msgs, idxs, idxb = _build_sc_gather()(msg, order, index.astype(jnp.int32))
    xpad = _tc_scan(msgs, idxb, W, b.reshape(1, D))
    return _build_sc_final()(idxs, xpad)


# speedup vs baseline: 1.0444x; 1.0444x over previous
import functools

import jax
import jax.numpy as jnp
from jax import lax
from jax.experimental import pallas as pl
from jax.experimental.pallas import tpu as pltpu
from jax.experimental.pallas import tpu_sc as plsc

N = 8192
D = 128
DIM = 2048
NC = 2
NS = 16
NW = NC * NS
ROWS_PER_TILE = N // NW
NODES_PER_TILE = DIM // NW
PAD_ROWS = 8
ZERO_ROW = N
NSTAGE = 13
LANES = 16

_PREC = jax.lax.Precision.DEFAULT


def _dot_t(a, m):
    return jax.lax.dot_general(a, m, (((1,), (1,)), ((), ())),
                               preferred_element_type=jnp.float32,
                               precision=_PREC)


SHIFT_PAD = N // 2


def _tc_scan_body(msgs_ref, idxp_ref, w_ref, b_ref, xpad_ref,
                  xbuf, ibuf, m_scr):
    w = w_ref[...]
    m_scr[...] = w
    ibuf[pl.ds(0, SHIFT_PAD), :] = jnp.full((SHIFT_PAD, D), -1.0, jnp.float32)
    ibuf[pl.ds(SHIFT_PAD, N), :] = idxp_ref[...]
    xbuf[0, pl.ds(SHIFT_PAD, N), :] = _dot_t(msgs_ref[...], w) + b_ref[...]

    def _cond(carry):
        k, p, cont = carry
        return (k < NSTAGE) & (cont == 1)

    def _stage(carry):
        k, p, cont = carry
        q = 1 - p
        s = jax.lax.shift_left(1, k)
        m = m_scr[...]
        d = ibuf[pl.ds(SHIFT_PAD, N), :] - ibuf[pl.ds(SHIFT_PAD - s, N), :]
        mask = jnp.abs(d) < 0.5
        x = xbuf[p, pl.ds(SHIFT_PAD, N), :]
        xs = xbuf[p, pl.ds(SHIFT_PAD - s, N), :]
        xm = jnp.where(mask, xs, jnp.zeros_like(xs))
        xbuf[q, pl.ds(SHIFT_PAD, N), :] = x + _dot_t(xm, m)
        m_scr[...] = jax.lax.dot_general(m, m, (((1,), (0,)), ((), ())),
                                         preferred_element_type=jnp.float32,
                                         precision=_PREC)
        return k + 1, q, jnp.max(jnp.where(mask, 1, 0))

    _, q, _ = jax.lax.while_loop(_cond, _stage, (0, 0, 1))
    xpad_ref[pl.ds(0, N), :] = xbuf[q, pl.ds(SHIFT_PAD, N), :]
    xpad_ref[pl.ds(N, PAD_ROWS), :] = jnp.zeros((PAD_ROWS, D), jnp.float32)


def _make_tc_scan(interpret=False):
  return pl.pallas_call(
    _tc_scan_body,
    interpret=interpret,
    out_shape=jax.ShapeDtypeStruct((N + PAD_ROWS, D), jnp.float32),
    scratch_shapes=[
        pltpu.VMEM((2, SHIFT_PAD + N, D), jnp.float32),
        pltpu.VMEM((SHIFT_PAD + N, D), jnp.float32),
        pltpu.VMEM((D, D), jnp.float32),
    ],
  )


_tc_scan = _make_tc_scan()


@functools.lru_cache(maxsize=None)
def _build_sc_gather():
    mesh = plsc.VectorSubcoreMesh(core_axis_name="c", subcore_axis_name="s")
    return functools.partial(
        pl.kernel,
        out_type=[
            jax.ShapeDtypeStruct((N, D), jnp.float32),
            jax.ShapeDtypeStruct((N,), jnp.int32),
        ],
        mesh=mesh,
        scratch_types=[
            pltpu.VMEM((ROWS_PER_TILE,), jnp.int32),
            pltpu.VMEM((ROWS_PER_TILE, D), jnp.float32),
            pltpu.VMEM((N,), jnp.int32),
            pltpu.VMEM((ROWS_PER_TILE,), jnp.int32),
            pltpu.SemaphoreType.DMA,
        ],
        compiler_params=pltpu.CompilerParams(needs_layout_passes=False),
    )(_sc_gather_body)


def _sc_gather_body(msg_hbm, order_hbm, index_hbm, msgs_hbm, idxs_hbm,
                    ord_v, rows_v, index_v, idxs_v, sem):
    wid = lax.axis_index("s") * NC + lax.axis_index("c")
    base = wid * ROWS_PER_TILE
    pltpu.sync_copy(order_hbm.at[pl.ds(base, ROWS_PER_TILE)], ord_v)
    copies = []
    for j in range(ROWS_PER_TILE // 128):
        copies.append(pltpu.async_copy(
            msg_hbm.at[ord_v.at[pl.ds(j * 128, 128)]],
            rows_v.at[pl.ds(j * 128, 128)], sem))
    pltpu.sync_copy(index_hbm, index_v)
    for g in range(ROWS_PER_TILE // LANES):
        o16 = ord_v[pl.ds(g * LANES, LANES)]
        idxs_v[pl.ds(g * LANES, LANES)] = plsc.load_gather(index_v, [o16])
    for c in copies:
        c.wait()
    pltpu.sync_copy(rows_v, msgs_hbm.at[pl.ds(base, ROWS_PER_TILE)])
    pltpu.sync_copy(idxs_v, idxs_hbm.at[pl.ds(base, ROWS_PER_TILE)])


@functools.lru_cache(maxsize=None)
def _build_sc_final():
    mesh = plsc.VectorSubcoreMesh(core_axis_name="c", subcore_axis_name="s")
    return functools.partial(
        pl.kernel,
        out_type=jax.ShapeDtypeStruct((DIM, D), jnp.float32),
        mesh=mesh,
        scratch_types=[
            pltpu.VMEM((N,), jnp.int32),
            pltpu.VMEM((NODES_PER_TILE,), jnp.int32),
            pltpu.VMEM((NODES_PER_TILE, D), jnp.float32),
            pltpu.SemaphoreType.DMA,
        ],
        compiler_params=pltpu.CompilerParams(needs_layout_passes=False),
    )(_sc_final_body)


def _sc_final_body(idxs_hbm, xpad_hbm, out_hbm, idx_v, fin_v, rows_v, sem):
    wid = lax.axis_index("s") * NC + lax.axis_index("c")
    base = wid * NODES_PER_TILE
    pltpu.sync_copy(idxs_hbm, idx_v)
    lane = lax.iota(jnp.int32, LANES)
    for g in range(NODES_PER_TILE // LANES):
        nid = base + g * LANES + lane
        lo = jnp.zeros((LANES,), jnp.int32)
        hi = jnp.full((LANES,), N, jnp.int32)
        for _ in range(NSTAGE):
            active = lo < hi
            mid = (lo + hi) >> 1
            midc = jnp.minimum(mid, jnp.full((LANES,), N - 1, jnp.int32))
            v = plsc.load_gather(idx_v, [midc])
            le = (v <= nid) & active
            gt = (v > nid) & active
            lo = jnp.where(le, mid + 1, lo)
            hi = jnp.where(gt, mid, hi)
        lp = lo - 1
        val = plsc.load_gather(idx_v, [jnp.maximum(lp, 0)])
        has = (lp >= 0) & (val == nid)
        fin_v[pl.ds(g * LANES, LANES)] = jnp.where(
            has, lp, jnp.full((LANES,), ZERO_ROW, jnp.int32))
    pltpu.async_copy(xpad_hbm.at[fin_v], rows_v, sem).wait()
    pltpu.sync_copy(rows_v, out_hbm.at[pl.ds(base, NODES_PER_TILE)])


def kernel(msg, index, t, dim_size, W, b):
    del dim_size
    order = jnp.lexsort((t, index)).astype(jnp.int32)
    msgs, idxs = _build_sc_gather()(msg, order, index.astype(jnp.int32))
    idxp = jnp.broadcast_to(idxs.astype(jnp.float32)[:, None], (N, D))
    xpad = _tc_scan(msgs, idxp, W, b.reshape(1, D))
    return _build_sc_final()(idxs, xpad)


# speedup vs baseline: 1.0521x; 1.0074x over previous
import functools

import jax
import jax.numpy as jnp
from jax import lax
from jax.experimental import pallas as pl
from jax.experimental.pallas import tpu as pltpu
from jax.experimental.pallas import tpu_sc as plsc

N = 8192
D = 128
DIM = 2048
NC = 2
NS = 16
NW = NC * NS
ROWS_PER_TILE = N // NW
NODES_PER_TILE = DIM // NW
PAD_ROWS = 8
ZERO_ROW = N
NSTAGE = 13
LANES = 16

_PREC = jax.lax.Precision.DEFAULT


def _dot_t(a, m):
    return jax.lax.dot_general(a, m, (((1,), (1,)), ((), ())),
                               preferred_element_type=jnp.float32,
                               precision=_PREC)


SHIFT_PAD = N // 2


def _tc_scan_body(msgs_ref, idxp_ref, w_ref, b_ref, xpad_ref,
                  xbuf, ibuf, m_scr):
    w = w_ref[...]
    m_scr[...] = w
    ibuf[pl.ds(0, SHIFT_PAD), :] = jnp.full((SHIFT_PAD, D), -1.0, jnp.float32)
    ibuf[pl.ds(SHIFT_PAD, N), :] = idxp_ref[...]
    xbuf[0, pl.ds(SHIFT_PAD, N), :] = _dot_t(msgs_ref[...], w) + b_ref[...]

    def _cond(carry):
        k, p, cont = carry
        return (k < NSTAGE) & (cont == 1)

    def _stage(carry):
        k, p, cont = carry
        q = 1 - p
        s = jax.lax.shift_left(1, k)
        m = m_scr[...]
        d = ibuf[pl.ds(SHIFT_PAD, N), :] - ibuf[pl.ds(SHIFT_PAD - s, N), :]
        mask = jnp.abs(d) < 0.5
        x = xbuf[p, pl.ds(SHIFT_PAD, N), :]
        xs = xbuf[p, pl.ds(SHIFT_PAD - s, N), :]
        xm = jnp.where(mask, xs, jnp.zeros_like(xs))
        xbuf[q, pl.ds(SHIFT_PAD, N), :] = x + _dot_t(xm, m)
        m_scr[...] = jax.lax.dot_general(m, m, (((1,), (0,)), ((), ())),
                                         preferred_element_type=jnp.float32,
                                         precision=_PREC)
        return k + 1, q, jnp.max(jnp.where(mask, 1, 0))

    _, q, _ = jax.lax.while_loop(_cond, _stage, (0, 0, 1))
    xpad_ref[pl.ds(0, N), :] = xbuf[q, pl.ds(SHIFT_PAD, N), :]
    xpad_ref[pl.ds(N, PAD_ROWS), :] = jnp.zeros((PAD_ROWS, D), jnp.float32)


def _make_tc_scan(interpret=False):
  return pl.pallas_call(
    _tc_scan_body,
    interpret=interpret,
    out_shape=jax.ShapeDtypeStruct((N + PAD_ROWS, D), jnp.float32),
    scratch_shapes=[
        pltpu.VMEM((2, SHIFT_PAD + N, D), jnp.float32),
        pltpu.VMEM((SHIFT_PAD + N, D), jnp.float32),
        pltpu.VMEM((D, D), jnp.float32),
    ],
  )


_tc_scan = _make_tc_scan()


@functools.lru_cache(maxsize=None)
def _build_sc_gather():
    mesh = plsc.VectorSubcoreMesh(core_axis_name="c", subcore_axis_name="s")
    return functools.partial(
        pl.kernel,
        out_type=[
            jax.ShapeDtypeStruct((N, D), jnp.float32),
            jax.ShapeDtypeStruct((N,), jnp.int32),
        ],
        mesh=mesh,
        scratch_types=[
            pltpu.VMEM((ROWS_PER_TILE,), jnp.int32),
            pltpu.VMEM((ROWS_PER_TILE, D), jnp.float32),
            pltpu.VMEM((N,), jnp.int32),
            pltpu.VMEM((ROWS_PER_TILE,), jnp.int32),
            pltpu.SemaphoreType.DMA,
        ],
        compiler_params=pltpu.CompilerParams(needs_layout_passes=False),
    )(_sc_gather_body)


def _sc_gather_body(msg_hbm, order_hbm, index_hbm, msgs_hbm, idxs_hbm,
                    ord_v, rows_v, index_v, idxs_v, sem):
    wid = lax.axis_index("s") * NC + lax.axis_index("c")
    base = wid * ROWS_PER_TILE
    pltpu.sync_copy(order_hbm.at[pl.ds(base, ROWS_PER_TILE)], ord_v)
    copies = []
    for j in range(ROWS_PER_TILE // 128):
        copies.append(pltpu.async_copy(
            msg_hbm.at[ord_v.at[pl.ds(j * 128, 128)]],
            rows_v.at[pl.ds(j * 128, 128)], sem))
    pltpu.sync_copy(index_hbm, index_v)
    for g in range(ROWS_PER_TILE // LANES):
        o16 = ord_v[pl.ds(g * LANES, LANES)]
        idxs_v[pl.ds(g * LANES, LANES)] = plsc.load_gather(index_v, [o16])
    for c in copies:
        c.wait()
    pltpu.sync_copy(rows_v, msgs_hbm.at[pl.ds(base, ROWS_PER_TILE)])
    pltpu.sync_copy(idxs_v, idxs_hbm.at[pl.ds(base, ROWS_PER_TILE)])


@functools.lru_cache(maxsize=None)
def _build_sc_search():
    mesh = plsc.VectorSubcoreMesh(core_axis_name="c", subcore_axis_name="s")
    return functools.partial(
        pl.kernel,
        out_type=jax.ShapeDtypeStruct((DIM,), jnp.int32),
        mesh=mesh,
        scratch_types=[
            pltpu.VMEM((N,), jnp.int32),
            pltpu.VMEM((NODES_PER_TILE,), jnp.int32),
            pltpu.SemaphoreType.DMA,
        ],
        compiler_params=pltpu.CompilerParams(needs_layout_passes=False),
    )(_sc_search_body)


def _sc_search_body(idxs_hbm, fin_hbm, idx_v, fin_v, sem):
    wid = lax.axis_index("s") * NC + lax.axis_index("c")
    base = wid * NODES_PER_TILE
    pltpu.sync_copy(idxs_hbm, idx_v)
    lane = lax.iota(jnp.int32, LANES)
    for g in range(NODES_PER_TILE // LANES):
        nid = base + g * LANES + lane
        lo = jnp.zeros((LANES,), jnp.int32)
        hi = jnp.full((LANES,), N, jnp.int32)
        for _ in range(NSTAGE):
            active = lo < hi
            mid = (lo + hi) >> 1
            midc = jnp.minimum(mid, jnp.full((LANES,), N - 1, jnp.int32))
            v = plsc.load_gather(idx_v, [midc])
            le = (v <= nid) & active
            gt = (v > nid) & active
            lo = jnp.where(le, mid + 1, lo)
            hi = jnp.where(gt, mid, hi)
        lp = lo - 1
        val = plsc.load_gather(idx_v, [jnp.maximum(lp, 0)])
        has = (lp >= 0) & (val == nid)
        fin_v[pl.ds(g * LANES, LANES)] = jnp.where(
            has, lp, jnp.full((LANES,), ZERO_ROW, jnp.int32))
    pltpu.sync_copy(fin_v, fin_hbm.at[pl.ds(base, NODES_PER_TILE)])


@functools.lru_cache(maxsize=None)
def _build_sc_final():
    mesh = plsc.VectorSubcoreMesh(core_axis_name="c", subcore_axis_name="s")
    return functools.partial(
        pl.kernel,
        out_type=jax.ShapeDtypeStruct((DIM, D), jnp.float32),
        mesh=mesh,
        scratch_types=[
            pltpu.VMEM((NODES_PER_TILE,), jnp.int32),
            pltpu.VMEM((NODES_PER_TILE, D), jnp.float32),
            pltpu.SemaphoreType.DMA,
        ],
        compiler_params=pltpu.CompilerParams(needs_layout_passes=False),
    )(_sc_final_body)


def _sc_final_body(fin_hbm, xpad_hbm, out_hbm, fin_v, rows_v, sem):
    wid = lax.axis_index("s") * NC + lax.axis_index("c")
    base = wid * NODES_PER_TILE
    pltpu.sync_copy(fin_hbm.at[pl.ds(base, NODES_PER_TILE)], fin_v)
    pltpu.async_copy(xpad_hbm.at[fin_v], rows_v, sem).wait()
    pltpu.sync_copy(rows_v, out_hbm.at[pl.ds(base, NODES_PER_TILE)])


def kernel(msg, index, t, dim_size, W, b):
    del dim_size
    order = jnp.lexsort((t, index)).astype(jnp.int32)
    msgs, idxs = _build_sc_gather()(msg, order, index.astype(jnp.int32))
    idxp = jnp.broadcast_to(idxs.astype(jnp.float32)[:, None], (N, D))
    fin = _build_sc_search()(idxs)
    xpad = _tc_scan(msgs, idxp, W, b.reshape(1, D))
    return _build_sc_final()(fin, xpad)
